# baseline (speedup 1.0000x reference)
import jax
import jax.numpy as jnp
from jax.experimental import pallas as pl

_TIME = 288
_F = 64
_T = 12
_NB = 128


def _body(xd_ref, xw_ref, tdT_ref, twT_ref, out_ref):
    dayv = xd_ref[0, 0, :, _T - 1 : _T]
    wkv = xw_ref[0, 0, :, _T - 1 : _T]
    di = jnp.clip((dayv * _TIME).astype(jnp.int32), 0, _TIME - 1)
    wi = jnp.clip(wkv.astype(jnp.int32), 0, 6)

    ohd = (di == jax.lax.broadcasted_iota(jnp.int32, (_NB, _TIME), 1)).astype(jnp.float32)
    ohw = (wi == jax.lax.broadcasted_iota(jnp.int32, (_NB, 8), 1)).astype(jnp.float32)

    dn = (((1,), (1,)), ((), ()))
    emb = jax.lax.dot_general(tdT_ref[...], ohd, dn, preferred_element_type=jnp.float32)
    emb = emb + jax.lax.dot_general(twT_ref[...], ohw, dn, preferred_element_type=jnp.float32)

    out_ref[0] = jnp.broadcast_to(emb[:, :, None], (_F, _NB, _T))


def kernel(x, time_day, time_week):
    B, C, N, T = x.shape
    F = time_day.shape[1]
    tdT = time_day.T
    twT = jnp.pad(time_week.T, ((0, 0), (0, 1)))

    grid = (B, N // _NB)
    return pl.pallas_call(
        _body,
        grid=grid,
        in_specs=[
            pl.BlockSpec((1, 1, _NB, T), lambda b, n: (b, 1, n, 0)),
            pl.BlockSpec((1, 1, _NB, T), lambda b, n: (b, 2, n, 0)),
            pl.BlockSpec((F, _TIME), lambda b, n: (0, 0)),
            pl.BlockSpec((F, 8), lambda b, n: (0, 0)),
        ],
        out_specs=pl.BlockSpec((1, F, _NB, T), lambda b, n: (b, 0, n, 0)),
        out_shape=jax.ShapeDtypeStruct((B, F, N, T), jnp.float32),
    )(x, x, tdT, twT)


# speedup vs baseline: 2.4974x; 2.4974x over previous
import jax
import jax.numpy as jnp
from jax.experimental import pallas as pl

_TIME = 288
_F = 64
_T = 12
_NB = 128


def _body(xd_ref, xw_ref, tdT_ref, twT_ref, r_ref, out_ref):
    dayv = xd_ref[0]
    wkv = xw_ref[0]
    di = jnp.clip((dayv * _TIME).astype(jnp.int32), 0, _TIME - 1)
    wi = jnp.clip(wkv.astype(jnp.int32), 0, 6)

    ohd = (di == jax.lax.broadcasted_iota(jnp.int32, (_TIME, _NB), 0)).astype(jnp.float32)
    ohw = (wi == jax.lax.broadcasted_iota(jnp.int32, (8, _NB), 0)).astype(jnp.float32)

    dn = (((1,), (0,)), ((), ()))
    emb = jax.lax.dot_general(tdT_ref[...], ohd, dn, preferred_element_type=jnp.float32)
    emb = emb + jax.lax.dot_general(twT_ref[...], ohw, dn, preferred_element_type=jnp.float32)

    out_ref[0] = jax.lax.dot_general(emb, r_ref[...], dn, preferred_element_type=jnp.float32)


def kernel(x, time_day, time_week):
    B, C, N, T = x.shape
    F = time_day.shape[1]
    dayf = x[:, 1, :, T - 1].reshape(B, 1, N)
    wkf = x[:, 2, :, T - 1].reshape(B, 1, N)
    tdT = time_day.T
    twT = jnp.pad(time_week.T, ((0, 0), (0, 1)))
    r = (
        jax.lax.broadcasted_iota(jnp.int32, (_NB, _NB * T), 0)
        == jax.lax.broadcasted_iota(jnp.int32, (_NB, _NB * T), 1) // T
    ).astype(jnp.float32)

    grid = (B, N // _NB)
    out_flat = pl.pallas_call(
        _body,
        grid=grid,
        in_specs=[
            pl.BlockSpec((1, 1, _NB), lambda b, n: (b, 0, n)),
            pl.BlockSpec((1, 1, _NB), lambda b, n: (b, 0, n)),
            pl.BlockSpec((F, _TIME), lambda b, n: (0, 0)),
            pl.BlockSpec((F, 8), lambda b, n: (0, 0)),
            pl.BlockSpec((_NB, _NB * T), lambda b, n: (0, 0)),
        ],
        out_specs=pl.BlockSpec((1, F, _NB * T), lambda b, n: (b, 0, n)),
        out_shape=jax.ShapeDtypeStruct((B, F, N * T), jnp.float32),
    )(dayf, wkf, tdT, twT, r)
    return out_flat.reshape(B, F, N, T)


# speedup vs baseline: 18.7222x; 7.4966x over previous
import jax
import jax.numpy as jnp
from jax.experimental import pallas as pl

_TIME = 288
_F = 64
_T = 12
_NB = 512


def _body(xd_ref, xw_ref, tdT_ref, twT_ref, out_ref):
    dayv = xd_ref[0]
    wkv = xw_ref[0]
    di = jnp.clip((dayv * _TIME).astype(jnp.int32), 0, _TIME - 1)
    wi = jnp.clip(wkv.astype(jnp.int32), 0, 6)

    ohd = (di == jax.lax.broadcasted_iota(jnp.int32, (_TIME, _NB), 0)).astype(jnp.float32)
    ohw = (wi == jax.lax.broadcasted_iota(jnp.int32, (8, _NB), 0)).astype(jnp.float32)

    dn = (((1,), (0,)), ((), ()))
    emb = jax.lax.dot_general(tdT_ref[...], ohd, dn, preferred_element_type=jnp.float32)
    emb = emb + jax.lax.dot_general(twT_ref[...], ohw, dn, preferred_element_type=jnp.float32)

    out_ref[0] = jnp.broadcast_to(emb[None], (_T, _F, _NB))


def kernel(x, time_day, time_week):
    B, C, N, T = x.shape
    F = time_day.shape[1]
    dayf = x[:, 1, :, T - 1].reshape(B, 1, N)
    wkf = x[:, 2, :, T - 1].reshape(B, 1, N)
    tdT = time_day.T
    twT = jnp.pad(time_week.T, ((0, 0), (0, 1)))

    grid = (B, N // _NB)
    out_tfn = pl.pallas_call(
        _body,
        grid=grid,
        in_specs=[
            pl.BlockSpec((1, 1, _NB), lambda b, n: (b, 0, n)),
            pl.BlockSpec((1, 1, _NB), lambda b, n: (b, 0, n)),
            pl.BlockSpec((F, _TIME), lambda b, n: (0, 0)),
            pl.BlockSpec((F, 8), lambda b, n: (0, 0)),
        ],
        out_specs=pl.BlockSpec((1, T, F, _NB), lambda b, n: (b, 0, 0, n)),
        out_shape=jax.ShapeDtypeStruct((B, T, F, N), jnp.float32),
    )(dayf, wkf, tdT, twT)
    return out_tfn.transpose(0, 2, 3, 1)
